# baseline (speedup 1.0000x reference)
import functools
import math

import jax
import jax.numpy as jnp
from jax import lax
from jax.experimental import pallas as pl
from jax.experimental.pallas import tpu as pltpu, tpu_sc as plsc

D_MODEL = 64
SCALE = math.sqrt(D_MODEL)
NUM_LANES = 16


@functools.partial(jax.jit, static_argnames=("chunk",))
def _gather_scaled(table, idx, chunk=1024):
    B = idx.shape[0]
    V, D = table.shape

    info = plsc.get_sparse_core_info()
    nw = info.num_cores * info.num_subcores
    assert B % (nw * chunk) == 0
    b_per_w = B // nw
    n_chunks = b_per_w // chunk

    mesh = plsc.VectorSubcoreMesh(core_axis_name="c", subcore_axis_name="s")

    @functools.partial(
        pl.kernel,
        mesh=mesh,
        out_type=jax.ShapeDtypeStruct((B, D), jnp.float32),
        scratch_types=[
            pltpu.VMEM((chunk,), jnp.int32),
            pltpu.VMEM((chunk, D), jnp.float32),
            pltpu.SemaphoreType.DMA,
        ],
        compiler_params=pltpu.CompilerParams(use_tc_tiling_on_sc=False),
    )
    def k(table_hbm, idx_hbm, out_hbm, idx_v, rows_v, sem):
        wid = lax.axis_index("s") * info.num_cores + lax.axis_index("c")
        base = wid * b_per_w

        def do_chunk(g, _):
            off = base + g * chunk
            pltpu.sync_copy(idx_hbm.at[pl.ds(off, chunk)], idx_v)
            pltpu.async_copy(table_hbm.at[idx_v], rows_v, sem).wait()

            def scale_row(i, _):
                for j in range(D // NUM_LANES):
                    sl = pl.ds(j * NUM_LANES, NUM_LANES)
                    rows_v[i, sl] = rows_v[i, sl] * SCALE
                return 0

            lax.fori_loop(0, chunk, scale_row, 0, unroll=4)
            pltpu.sync_copy(rows_v, out_hbm.at[pl.ds(off, chunk)])
            return 0

        lax.fori_loop(0, n_chunks, do_chunk, 0)

    return k(table, idx)


def kernel(x, table):
    B0, B1 = x.shape
    idx = x.reshape(B0 * B1).astype(jnp.int32)
    out = _gather_scaled(table, idx)
    return out.reshape(B0, B1, D_MODEL)


# speedup vs baseline: 1.0625x; 1.0625x over previous
import functools
import math

import jax
import jax.numpy as jnp
from jax import lax
from jax.experimental import pallas as pl
from jax.experimental.pallas import tpu as pltpu, tpu_sc as plsc

D_MODEL = 64
SCALE = math.sqrt(D_MODEL)
NUM_LANES = 16


@functools.partial(jax.jit, static_argnames=("chunk",))
def _gather_scaled(table, idx, chunk=800):
    B = idx.shape[0]
    V, D = table.shape

    info = plsc.get_sparse_core_info()
    nw = info.num_cores * info.num_subcores
    assert B % (nw * chunk) == 0
    b_per_w = B // nw
    n_chunks = b_per_w // chunk
    assert n_chunks % 2 == 0

    mesh = plsc.VectorSubcoreMesh(core_axis_name="c", subcore_axis_name="s")

    @functools.partial(
        pl.kernel,
        mesh=mesh,
        out_type=jax.ShapeDtypeStruct((B, D), jnp.float32),
        scratch_types=[
            pltpu.VMEM((b_per_w,), jnp.int32),
            pltpu.VMEM((2, chunk, D), jnp.float32),
            pltpu.SemaphoreType.DMA((2,)),
            pltpu.SemaphoreType.DMA((2,)),
        ],
        compiler_params=pltpu.CompilerParams(use_tc_tiling_on_sc=False),
    )
    def k(table_hbm, idx_hbm, out_hbm, idx_v, rows_v, gsem, ssem):
        wid = lax.axis_index("s") * info.num_cores + lax.axis_index("c")
        base = wid * b_per_w

        pltpu.sync_copy(idx_hbm.at[pl.ds(base, b_per_w)], idx_v)

        def gather(g, b):
            off = pl.multiple_of(g * chunk, 8)
            pltpu.async_copy(
                table_hbm.at[idx_v.at[pl.ds(off, chunk)]],
                rows_v.at[b],
                gsem.at[b],
            )

        def wait_gather(b):
            pltpu.make_async_copy(
                table_hbm.at[idx_v.at[pl.ds(0, chunk)]],
                rows_v.at[b],
                gsem.at[b],
            ).wait()

        def scatter(g, b):
            off = pl.multiple_of(base + g * chunk, 8)
            pltpu.async_copy(
                rows_v.at[b],
                out_hbm.at[pl.ds(off, chunk)],
                ssem.at[b],
            )

        def wait_scatter(b):
            pltpu.make_async_copy(
                rows_v.at[b],
                out_hbm.at[pl.ds(base, chunk)],
                ssem.at[b],
            ).wait()

        gather(0, 0)

        def pair(i, _):
            for b in range(2):
                g = 2 * i + b
                bn = 1 - b
                wait_gather(b)

                @pl.when(g >= 1)
                def _():
                    wait_scatter(bn)

                @pl.when(g + 1 < n_chunks)
                def _():
                    gather(g + 1, bn)

                def scale_row(r, _):
                    for j in range(D // NUM_LANES):
                        sl = pl.ds(j * NUM_LANES, NUM_LANES)
                        rows_v[b, r, sl] = rows_v[b, r, sl] * SCALE
                    return 0

                lax.fori_loop(0, chunk, scale_row, 0, unroll=4)
                scatter(g, b)
            return 0

        lax.fori_loop(0, n_chunks // 2, pair, 0)
        wait_scatter((n_chunks - 1) % 2)

    return k(table, idx)


def kernel(x, table):
    B0, B1 = x.shape
    idx = x.reshape(B0 * B1).astype(jnp.int32)
    out = _gather_scaled(idx=idx, table=table)
    return out.reshape(B0, B1, D_MODEL)
